# baseline (speedup 1.0000x reference)
import functools

import jax
import jax.numpy as jnp
from jax import lax
from jax.experimental import pallas as pl
from jax.experimental.pallas import tpu as pltpu, tpu_sc as plsc

_info = plsc.get_sparse_core_info()
_NC = _info.num_cores
_NS = _info.num_subcores
_NW = _NC * _NS

_G = 16


def _make_gather(B, D):
  b_per_w = B // _NW
  half = b_per_w // 2
  n_groups = half // _G
  mesh = plsc.VectorSubcoreMesh(core_axis_name="c", subcore_axis_name="s")

  @functools.partial(
      pl.kernel,
      mesh=mesh,
      out_type=jax.ShapeDtypeStruct((B, D), jnp.float32),
      scratch_types=[
          pltpu.VMEM((b_per_w,), jnp.int32),
          pltpu.VMEM((half, D), jnp.float32),
          pltpu.VMEM_SHARED((_NS, half, D), jnp.float32),
          pltpu.SemaphoreType.DMA,
          pltpu.SemaphoreType.DMA,
      ],
  )
  def gather_kernel(table_hbm, idx_hbm, out_hbm, idx_v, rows_v, shr_v, sem,
                    sem2):
    wid = lax.axis_index("s") * _NC + lax.axis_index("c")
    sid = lax.axis_index("s")
    base = wid * b_per_w
    pltpu.sync_copy(idx_hbm.at[pl.ds(base, b_per_w)], idx_v)

    def fire(g, _):
      vec_a = idx_v[pl.ds(g * _G, _G)]
      vec_b = idx_v[pl.ds(half + g * _G, _G)]
      for k in range(_G):
        pltpu.async_copy(table_hbm.at[vec_a[k]], rows_v.at[g * _G + k], sem)
        pltpu.async_copy(table_hbm.at[vec_b[k]],
                         shr_v.at[sid, g * _G + k], sem2)
      return 0

    lax.fori_loop(0, n_groups, fire, 0)

    def drain(g, _):
      for k in range(_G):
        pltpu.make_async_copy(table_hbm.at[0], rows_v.at[0], sem).wait()
        pltpu.make_async_copy(table_hbm.at[0], shr_v.at[sid, 0], sem2).wait()
      return 0

    lax.fori_loop(0, n_groups, drain, 0)
    pltpu.sync_copy(rows_v, out_hbm.at[pl.ds(base, half)])
    pltpu.sync_copy(shr_v.at[sid], out_hbm.at[pl.ds(base + half, half)])

  return gather_kernel


@jax.jit
def kernel(users, U_g):
  flat = users.reshape(-1).astype(jnp.int32)
  out = _make_gather(flat.shape[0], U_g.shape[1])(U_g, flat)
  return out.reshape(tuple(users.shape) + (U_g.shape[1],))


# speedup vs baseline: 1.0381x; 1.0381x over previous
import functools

import jax
import jax.numpy as jnp
from jax import lax
from jax.experimental import pallas as pl
from jax.experimental.pallas import tpu as pltpu, tpu_sc as plsc

_info = plsc.get_sparse_core_info()
_NC = _info.num_cores
_NS = _info.num_subcores
_NW = _NC * _NS

_G = 16


def _make_gather(B, D):
  b_per_w = B // _NW
  n_groups = b_per_w // _G
  mesh = plsc.VectorSubcoreMesh(core_axis_name="c", subcore_axis_name="s")

  @functools.partial(
      pl.kernel,
      mesh=mesh,
      out_type=jax.ShapeDtypeStruct((B, D), jnp.float32),
      scratch_types=[
          pltpu.VMEM((b_per_w,), jnp.int32),
          pltpu.VMEM((b_per_w, D), jnp.float32),
          pltpu.SemaphoreType.DMA,
      ],
  )
  def gather_kernel(table_hbm, idx_hbm, out_hbm, idx_v, rows_v, sem):
    wid = lax.axis_index("s") * _NC + lax.axis_index("c")
    base = wid * b_per_w
    pltpu.sync_copy(idx_hbm.at[pl.ds(base, b_per_w)], idx_v)

    def fire(g, _):
      vec = idx_v[pl.ds(g * _G, _G)]
      for k in range(_G):
        pltpu.async_copy(table_hbm.at[vec[k]], rows_v.at[g * _G + k], sem)
      return 0

    lax.fori_loop(0, n_groups, fire, 0)

    def drain(g, _):
      for k in range(_G):
        pltpu.make_async_copy(table_hbm.at[0], rows_v.at[0], sem).wait()
      return 0

    lax.fori_loop(0, n_groups, drain, 0)
    pltpu.sync_copy(rows_v, out_hbm.at[pl.ds(base, b_per_w)])

  return gather_kernel


@jax.jit
def kernel(users, U_g):
  flat = users.reshape(-1).astype(jnp.int32)
  out = _make_gather(flat.shape[0], U_g.shape[1])(U_g, flat)
  return out.reshape(tuple(users.shape) + (U_g.shape[1],))
